# Initial kernel scaffold; baseline (speedup 1.0000x reference)
import jax
import jax.numpy as jnp
from jax.experimental import pallas as pl


def kernel(input_repr, W_in1, b_in1, ln_in_w, ln_in_b, Wp1, bp1, Wp2, bp2, lnq_w, lnq_b, Wg, bg, attn_in_w, attn_in_b, attn_out_w, attn_out_b, Wt1, bt1, lnt_w, lnt_b, Wt2, bt2, Wc1, bc1, Wc2, bc2, Wa1, ba1, Wa2, ba2, corpus_keys, corpus_values):
    raise NotImplementedError("write your pallas kernel here")


# speedup vs baseline: 1.9181x; 1.9181x over previous
import functools
import jax
import jax.numpy as jnp
import numpy as np
from jax import lax
from jax.experimental import pallas as pl
from jax.experimental.pallas import tpu as pltpu
from jax.experimental.pallas import tpu_sc as plsc

_D = 512
_QD = 128
_NH = 8
_DH = _D // _NH
_K = 3
_STEPS = 5
_C = 100000
_V = 50257
_B = 1024

_CB = 2000
_NCB = _C // _CB

_SC_NC = 2
_SC_NS = 16
_NW = _SC_NC * _SC_NS
_NROWS = _B * _K
_RPW = _NROWS // _NW


def _gelu(x):
    return x * (lax.erf(x / np.sqrt(2).astype(np.float32)) + 1) / 2


def _ln(x, w, b, eps=1e-5):
    m = jnp.mean(x, axis=-1, keepdims=True)
    v = jnp.mean((x - m) * (x - m), axis=-1, keepdims=True)
    return (x - m) / jnp.sqrt(v + eps) * w + b


def _nrm(x, eps=1e-12):
    n = jnp.sqrt(jnp.sum(x * x, axis=-1, keepdims=True))
    return x / jnp.maximum(n, eps)


def _dott(a, w):
    return lax.dot_general(a, w, (((1,), (1,)), ((), ())),
                           preferred_element_type=jnp.float32)


def _input_body(x_ref, w_ref, b_ref, lw_ref, lb_ref, o_ref):
    h = _dott(x_ref[...], w_ref[...]) + b_ref[...]
    o_ref[...] = _ln(_gelu(h), lw_ref[...], lb_ref[...])


def _input_call(x, w, b, lw, lb):
    return pl.pallas_call(
        _input_body,
        out_shape=jax.ShapeDtypeStruct((_B, _D), jnp.float32),
    )(x, w, b.reshape(1, _D), lw.reshape(1, _D), lb.reshape(1, _D))


def _cnorm_body(x_ref, o_ref):
    o_ref[...] = _nrm(x_ref[...])


def _cnorm_call(ck):
    blk = 2000
    return pl.pallas_call(
        _cnorm_body,
        grid=(_C // blk,),
        in_specs=[pl.BlockSpec((blk, _QD), lambda c: (c, 0))],
        out_specs=pl.BlockSpec((blk, _QD), lambda c: (c, 0)),
        out_shape=jax.ShapeDtypeStruct((_C, _QD), jnp.float32),
    )(ck)


def _qgen_body(with_gate, cur_ref, ctx_ref, wp1_ref, bp1_ref, wp2_ref,
               bp2_ref, lw_ref, lb_ref, wg_ref, bg_ref, o_ref):
    cur = cur_ref[...]
    h = _gelu(_dott(cur, wp1_ref[...]) + bp1_ref[...])
    q = _ln(_dott(h, wp2_ref[...]) + bp2_ref[...], lw_ref[...], lb_ref[...])
    if with_gate:
        cc = jnp.concatenate([cur, ctx_ref[...]], axis=1)
        g = jax.nn.sigmoid(_dott(cc, wg_ref[...]) + bg_ref[...])
        q = q * g
    q = _nrm(q)
    q = _nrm(q)
    o_ref[...] = q


def _qgen_call(with_gate, cur, ctx, wp1, bp1, wp2, bp2, lqw, lqb, wg, bg):
    body = functools.partial(_qgen_body, with_gate)
    return pl.pallas_call(
        body,
        out_shape=jax.ShapeDtypeStruct((_B, _QD), jnp.float32),
    )(cur, ctx, wp1, bp1.reshape(1, _D), wp2, bp2.reshape(1, _QD),
      lqw.reshape(1, _QD), lqb.reshape(1, _QD), wg, bg.reshape(1, _QD))


def _simtop_body(q_ref, ck_ref, oi_ref, bs_ref, bi_ref):
    c = pl.program_id(0)
    s = _dott(q_ref[...], ck_ref[...])
    ii = lax.broadcasted_iota(jnp.int32, (_B, _CB), 1)
    ms = []
    mi = []
    for _ in range(_K):
        m = jnp.max(s, axis=1)
        pos = jnp.min(jnp.where(s == m[:, None], ii, _CB), axis=1)
        ms.append(m[:, None])
        mi.append((pos + c * _CB)[:, None])
        s = jnp.where(ii == pos[:, None], -jnp.inf, s)
    blk_s = jnp.concatenate(ms, axis=1)
    blk_i = jnp.concatenate(mi, axis=1)

    @pl.when(c == 0)
    def _():
        bs_ref[...] = blk_s
        bi_ref[...] = blk_i

    @pl.when(c > 0)
    def _():
        cs = jnp.concatenate([bs_ref[...], blk_s], axis=1)
        ci = jnp.concatenate([bi_ref[...], blk_i], axis=1)
        i6 = lax.broadcasted_iota(jnp.int32, (_B, 2 * _K), 1)
        os_ = []
        oi_ = []
        for _j in range(_K):
            m = jnp.max(cs, axis=1)
            pos = jnp.min(jnp.where(cs == m[:, None], i6, 2 * _K), axis=1)
            sel = i6 == pos[:, None]
            os_.append(m[:, None])
            oi_.append(jnp.sum(jnp.where(sel, ci, 0), axis=1)[:, None])
            cs = jnp.where(sel, -jnp.inf, cs)
        bs_ref[...] = jnp.concatenate(os_, axis=1)
        bi_ref[...] = jnp.concatenate(oi_, axis=1)

    @pl.when(c == _NCB - 1)
    def _():
        oi_ref[...] = bi_ref[...]


def _simtop_call(q, corpus_norm):
    return pl.pallas_call(
        _simtop_body,
        grid=(_NCB,),
        in_specs=[
            pl.BlockSpec((_B, _QD), lambda c: (0, 0)),
            pl.BlockSpec((_CB, _QD), lambda c: (c, 0)),
        ],
        out_specs=pl.BlockSpec((_B, _K), lambda c: (0, 0)),
        out_shape=jax.ShapeDtypeStruct((_B, _K), jnp.int32),
        scratch_shapes=[
            pltpu.VMEM((_B, _K), jnp.float32),
            pltpu.VMEM((_B, _K), jnp.int32),
        ],
    )(q, corpus_norm)


@functools.cache
def _make_sc_gather():
    mesh = plsc.VectorSubcoreMesh(core_axis_name="c", subcore_axis_name="s")

    @functools.partial(
        pl.kernel, mesh=mesh,
        out_type=jax.ShapeDtypeStruct((_NROWS, _D), jnp.float32),
        scratch_types=[
            pltpu.VMEM((_RPW,), jnp.int32),
            pltpu.VMEM((_RPW, _D), jnp.float32),
            pltpu.SemaphoreType.DMA,
        ],
    )
    def sc_gather(table_hbm, idx_hbm, out_hbm, idx_v, rows_v, sem):
        wid = lax.axis_index("s") * _SC_NC + lax.axis_index("c")
        base = wid * _RPW
        pltpu.sync_copy(idx_hbm.at[pl.ds(base, _RPW)], idx_v)
        pltpu.async_copy(table_hbm.at[idx_v], rows_v, sem).wait()
        pltpu.sync_copy(rows_v, out_hbm.at[pl.ds(base, _RPW)])

    return sc_gather


def _sc_gather(table, idx):
    return _make_sc_gather()(table, idx)


_TB = 256


def _thought_body(first_step, cur_ref, docs_ref, ctx_ref, wq_ref, bq_ref,
                  wk_ref, bk_ref, wv_ref, bv_ref, wo_ref, bo_ref,
                  wt1_ref, bt1_ref, ltw_ref, ltb_ref,
                  wt2_ref, bt2_ref, oth_ref, octx_ref):
    cur = cur_ref[...]
    qp = _dott(cur, wq_ref[...]) + bq_ref[...]
    ks = []
    vs = []
    for k in range(_K):
        dk = docs_ref[:, k * _D:(k + 1) * _D]
        ks.append(_dott(dk, wk_ref[...]) + bk_ref[...])
        vs.append(_dott(dk, wv_ref[...]) + bv_ref[...])
    scale = np.float32(1.0 / np.sqrt(_DH))
    sks = []
    for k in range(_K):
        cols = []
        for h in range(_NH):
            sl = slice(h * _DH, (h + 1) * _DH)
            cols.append(jnp.sum(qp[:, sl] * ks[k][:, sl], axis=1,
                                keepdims=True) * scale)
        sks.append(jnp.concatenate(cols, axis=1))
    m = jnp.maximum(jnp.maximum(sks[0], sks[1]), sks[2])
    es = [jnp.exp(sk - m) for sk in sks]
    z = es[0] + es[1] + es[2]
    ws = [e / z for e in es]
    attn = None
    for k in range(_K):
        wf = jnp.concatenate(
            [lax.broadcast_in_dim(ws[k][:, h:h + 1], (_TB, _DH), (0, 1))
             for h in range(_NH)], axis=1)
        contrib = wf * vs[k]
        attn = contrib if attn is None else attn + contrib
    ao = _dott(attn, wo_ref[...]) + bo_ref[...]
    prev = cur if first_step else ctx_ref[...]
    combined = jnp.concatenate([ao, prev], axis=1)
    h1 = _ln(_gelu(_dott(combined, wt1_ref[...]) + bt1_ref[...]),
             ltw_ref[...], ltb_ref[...])
    th = _dott(h1, wt2_ref[...]) + bt2_ref[...]
    oth_ref[...] = th
    octx_ref[...] = th if first_step else 0.7 * th + 0.3 * ctx_ref[...]


def _thought_call(first_step, cur, docs, ctx, wq, bq, wk, bk, wv, bv,
                  wo, bo, wt1, bt1, ltw, ltb, wt2, bt2):
    body = functools.partial(_thought_body, first_step)
    row = pl.BlockSpec((_TB, _D), lambda i: (i, 0))
    drow = pl.BlockSpec((_TB, _K * _D), lambda i: (i, 0))
    wsq = pl.BlockSpec((_D, _D), lambda i: (0, 0))
    wt1s = pl.BlockSpec((_D, 2 * _D), lambda i: (0, 0))
    bsp = pl.BlockSpec((1, _D), lambda i: (0, 0))
    return pl.pallas_call(
        body,
        grid=(_B // _TB,),
        in_specs=[row, drow, row, wsq, bsp, wsq, bsp, wsq, bsp, wsq, bsp,
                  wt1s, bsp, bsp, bsp, wsq, bsp],
        out_specs=(row, row),
        out_shape=(
            jax.ShapeDtypeStruct((_B, _D), jnp.float32),
            jax.ShapeDtypeStruct((_B, _D), jnp.float32),
        ),
    )(cur, docs, ctx, wq, bq.reshape(1, _D), wk, bk.reshape(1, _D),
      wv, bv.reshape(1, _D), wo, bo.reshape(1, _D), wt1,
      bt1.reshape(1, _D), ltw.reshape(1, _D), ltb.reshape(1, _D),
      wt2, bt2.reshape(1, _D))


def _head1_body(x_ref, w_ref, b_ref, o_ref):
    o_ref[...] = _gelu(_dott(x_ref[...], w_ref[...]) + b_ref[...])


def _head1_call(cur, wa1, ba1):
    return pl.pallas_call(
        _head1_body,
        out_shape=jax.ShapeDtypeStruct((_B, _D), jnp.float32),
    )(cur, wa1, ba1.reshape(1, _D))


_VB = 4096
_NVB = (_V + _VB - 1) // _VB


def _head2_body(h_ref, w_ref, b_ref, o_ref):
    o_ref[...] = _dott(h_ref[...], w_ref[...]) + b_ref[...]


def _head2_call(h, wa2, ba2):
    return pl.pallas_call(
        _head2_body,
        grid=(_NVB,),
        in_specs=[
            pl.BlockSpec((_B, _D), lambda c: (0, 0)),
            pl.BlockSpec((_VB, _D), lambda c: (c, 0)),
            pl.BlockSpec((1, _VB), lambda c: (0, c)),
        ],
        out_specs=pl.BlockSpec((_B, _VB), lambda c: (0, c)),
        out_shape=jax.ShapeDtypeStruct((_B, _V), jnp.float32),
    )(h, wa2, ba2.reshape(1, _V))


def kernel(input_repr, W_in1, b_in1, ln_in_w, ln_in_b, Wp1, bp1, Wp2, bp2,
           lnq_w, lnq_b, Wg, bg, attn_in_w, attn_in_b, attn_out_w,
           attn_out_b, Wt1, bt1, lnt_w, lnt_b, Wt2, bt2, Wc1, bc1, Wc2,
           bc2, Wa1, ba1, Wa2, ba2, corpus_keys, corpus_values):
    Wq = attn_in_w[:_D]
    Wk = attn_in_w[_D:2 * _D]
    Wv = attn_in_w[2 * _D:]
    bq = attn_in_b[:_D]
    bk = attn_in_b[_D:2 * _D]
    bv = attn_in_b[2 * _D:]

    current = _input_call(input_repr, W_in1, b_in1, ln_in_w, ln_in_b)
    corpus_norm = _cnorm_call(corpus_keys)
    ctx = current
    for step in range(_STEPS):
        q = _qgen_call(step > 0, current, ctx, Wp1, bp1, Wp2, bp2,
                       lnq_w, lnq_b, Wg, bg)
        idx = _simtop_call(q, corpus_norm)
        docs = _sc_gather(corpus_values, idx.reshape(_NROWS))
        docs = docs.reshape(_B, _K * _D)
        current, ctx = _thought_call(step == 0, current, docs, ctx,
                                     Wq, bq, Wk, bk, Wv, bv,
                                     attn_out_w, attn_out_b, Wt1, bt1,
                                     lnt_w, lnt_b, Wt2, bt2)
    h = _head1_call(current, Wa1, ba1)
    return _head2_call(h, Wa2, ba2)


# speedup vs baseline: 2.9450x; 1.5354x over previous
import functools
import jax
import jax.numpy as jnp
import numpy as np
from jax import lax
from jax.experimental import pallas as pl
from jax.experimental.pallas import tpu as pltpu
from jax.experimental.pallas import tpu_sc as plsc

_D = 512
_QD = 128
_NH = 8
_DH = _D // _NH
_K = 3
_STEPS = 5
_C = 100000
_V = 50257
_B = 1024

_CB = 2048
_TILES = _CB // 128
_NCB = (_C + _CB - 1) // _CB
_CPAD = _NCB * _CB

_SC_NC = 2
_SC_NS = 16
_NW = _SC_NC * _SC_NS
_NROWS = _B * _K
_RPW = _NROWS // _NW


def _gelu(x):
    return x * (lax.erf(x / np.sqrt(2).astype(np.float32)) + 1) / 2


def _ln(x, w, b, eps=1e-5):
    m = jnp.mean(x, axis=-1, keepdims=True)
    v = jnp.mean((x - m) * (x - m), axis=-1, keepdims=True)
    return (x - m) / jnp.sqrt(v + eps) * w + b


def _nrm(x, eps=1e-12):
    n = jnp.sqrt(jnp.sum(x * x, axis=-1, keepdims=True))
    return x / jnp.maximum(n, eps)


def _dott(a, w):
    return lax.dot_general(a, w, (((1,), (1,)), ((), ())),
                           preferred_element_type=jnp.float32)


def _input_body(x_ref, w_ref, b_ref, lw_ref, lb_ref, o_ref):
    h = _dott(x_ref[...], w_ref[...]) + b_ref[...]
    o_ref[...] = _ln(_gelu(h), lw_ref[...], lb_ref[...])


def _input_call(x, w, b, lw, lb):
    return pl.pallas_call(
        _input_body,
        out_shape=jax.ShapeDtypeStruct((_B, _D), jnp.float32),
    )(x, w, b.reshape(1, _D), lw.reshape(1, _D), lb.reshape(1, _D))


def _cnorm_body(x_ref, o_ref):
    c = pl.program_id(0)
    y = _nrm(x_ref[...])
    rows = lax.broadcasted_iota(jnp.int32, (_CB, _QD), 0) + c * _CB
    o_ref[...] = jnp.where(rows < _C, y, 0.0)


def _cnorm_call(ck):
    return pl.pallas_call(
        _cnorm_body,
        grid=(_NCB,),
        in_specs=[pl.BlockSpec((_CB, _QD), lambda c: (c, 0))],
        out_specs=pl.BlockSpec((_CB, _QD), lambda c: (c, 0)),
        out_shape=jax.ShapeDtypeStruct((_CPAD, _QD), jnp.float32),
    )(ck)


def _qgen_body(with_gate, cur_ref, ctx_ref, wp1_ref, bp1_ref, wp2_ref,
               bp2_ref, lw_ref, lb_ref, wg_ref, bg_ref, o_ref):
    cur = cur_ref[...]
    h = _gelu(_dott(cur, wp1_ref[...]) + bp1_ref[...])
    q = _ln(_dott(h, wp2_ref[...]) + bp2_ref[...], lw_ref[...], lb_ref[...])
    if with_gate:
        cc = jnp.concatenate([cur, ctx_ref[...]], axis=1)
        g = jax.nn.sigmoid(_dott(cc, wg_ref[...]) + bg_ref[...])
        q = q * g
    q = _nrm(q)
    q = _nrm(q)
    o_ref[...] = q


def _qgen_call(with_gate, cur, ctx, wp1, bp1, wp2, bp2, lqw, lqb, wg, bg):
    body = functools.partial(_qgen_body, with_gate)
    return pl.pallas_call(
        body,
        out_shape=jax.ShapeDtypeStruct((_B, _QD), jnp.float32),
    )(cur, ctx, wp1, bp1.reshape(1, _D), wp2, bp2.reshape(1, _QD),
      lqw.reshape(1, _QD), lqb.reshape(1, _QD), wg, bg.reshape(1, _QD))


def _simtop_body(q_ref, ck_ref, pen_ref, oi_ref, t1r, t2r, t3r,
                 i1r, i2r, i3r):
    c = pl.program_id(0)

    @pl.when(c == 0)
    def _():
        neg = jnp.full((_B, 128), -jnp.inf, jnp.float32)
        zero = jnp.zeros((_B, 128), jnp.int32)
        t1r[...] = neg
        t2r[...] = neg
        t3r[...] = neg
        i1r[...] = zero
        i2r[...] = zero
        i3r[...] = zero

    q = q_ref[...]
    t1 = t1r[...]
    t2 = t2r[...]
    t3 = t3r[...]
    i1 = i1r[...]
    i2 = i2r[...]
    i3 = i3r[...]
    for t in range(_TILES):
        ck_t = ck_ref[t * 128:(t + 1) * 128, :]
        s = _dott(q, ck_t) + pen_ref[:, t * 128:(t + 1) * 128]
        tid = lax.broadcast_in_dim(c * _TILES + t, (_B, 128), ())
        c1 = s > t1
        sp1 = jnp.where(c1, t1, s)
        si1 = jnp.where(c1, i1, tid)
        t1 = jnp.where(c1, s, t1)
        i1 = jnp.where(c1, tid, i1)
        c2 = sp1 > t2
        sp2 = jnp.where(c2, t2, sp1)
        si2 = jnp.where(c2, i2, si1)
        t2 = jnp.where(c2, sp1, t2)
        i2 = jnp.where(c2, si1, i2)
        c3 = sp2 > t3
        t3 = jnp.where(c3, sp2, t3)
        i3 = jnp.where(c3, si2, i3)
    t1r[...] = t1
    t2r[...] = t2
    t3r[...] = t3
    i1r[...] = i1
    i2r[...] = i2
    i3r[...] = i3

    @pl.when(c == _NCB - 1)
    def _():
        li = lax.broadcasted_iota(jnp.int32, (_B, 128), 1)
        a1 = t1r[...]
        a2 = t2r[...]
        a3 = t3r[...]
        b1 = i1r[...]
        b2 = i2r[...]
        b3 = i3r[...]
        outs = []
        for _j in range(_K):
            m = jnp.max(a1, axis=1)
            pos = jnp.min(jnp.where(a1 == m[:, None], li, 128), axis=1)
            sel = li == pos[:, None]
            gid = jnp.sum(jnp.where(sel, b1, 0), axis=1) * 128 + pos
            outs.append(gid[:, None])
            a1 = jnp.where(sel, a2, a1)
            b1 = jnp.where(sel, b2, b1)
            a2 = jnp.where(sel, a3, a2)
            b2 = jnp.where(sel, b3, b2)
            a3 = jnp.where(sel, -jnp.inf, a3)
        oi_ref[...] = jnp.concatenate(outs, axis=1)


def _simtop_call(q, corpus_norm, penalty):
    return pl.pallas_call(
        _simtop_body,
        grid=(_NCB,),
        in_specs=[
            pl.BlockSpec((_B, _QD), lambda c: (0, 0)),
            pl.BlockSpec((_CB, _QD), lambda c: (c, 0)),
            pl.BlockSpec((1, _CB), lambda c: (0, c)),
        ],
        out_specs=pl.BlockSpec((_B, _K), lambda c: (0, 0)),
        out_shape=jax.ShapeDtypeStruct((_B, _K), jnp.int32),
        scratch_shapes=[
            pltpu.VMEM((_B, 128), jnp.float32),
            pltpu.VMEM((_B, 128), jnp.float32),
            pltpu.VMEM((_B, 128), jnp.float32),
            pltpu.VMEM((_B, 128), jnp.int32),
            pltpu.VMEM((_B, 128), jnp.int32),
            pltpu.VMEM((_B, 128), jnp.int32),
        ],
    )(q, corpus_norm, penalty)


@functools.cache
def _make_sc_gather():
    mesh = plsc.VectorSubcoreMesh(core_axis_name="c", subcore_axis_name="s")

    @functools.partial(
        pl.kernel, mesh=mesh,
        out_type=jax.ShapeDtypeStruct((_NROWS, _D), jnp.float32),
        scratch_types=[
            pltpu.VMEM((_RPW,), jnp.int32),
            pltpu.VMEM((_RPW, _D), jnp.float32),
            pltpu.SemaphoreType.DMA,
        ],
    )
    def sc_gather(table_hbm, idx_hbm, out_hbm, idx_v, rows_v, sem):
        wid = lax.axis_index("s") * _SC_NC + lax.axis_index("c")
        base = wid * _RPW
        pltpu.sync_copy(idx_hbm.at[pl.ds(base, _RPW)], idx_v)
        pltpu.async_copy(table_hbm.at[idx_v], rows_v, sem).wait()
        pltpu.sync_copy(rows_v, out_hbm.at[pl.ds(base, _RPW)])

    return sc_gather


def _sc_gather(table, idx):
    return _make_sc_gather()(table, idx)


_TB = 256


def _thought_body(first_step, cur_ref, docs_ref, ctx_ref, wq_ref, bq_ref,
                  wk_ref, bk_ref, wv_ref, bv_ref, wo_ref, bo_ref,
                  wt1_ref, bt1_ref, ltw_ref, ltb_ref,
                  wt2_ref, bt2_ref, oth_ref, octx_ref):
    cur = cur_ref[...]
    qp = _dott(cur, wq_ref[...]) + bq_ref[...]
    ks = []
    vs = []
    for k in range(_K):
        dk = docs_ref[:, k * _D:(k + 1) * _D]
        ks.append(_dott(dk, wk_ref[...]) + bk_ref[...])
        vs.append(_dott(dk, wv_ref[...]) + bv_ref[...])
    scale = np.float32(1.0 / np.sqrt(_DH))
    sks = []
    for k in range(_K):
        cols = []
        for h in range(_NH):
            sl = slice(h * _DH, (h + 1) * _DH)
            cols.append(jnp.sum(qp[:, sl] * ks[k][:, sl], axis=1,
                                keepdims=True) * scale)
        sks.append(jnp.concatenate(cols, axis=1))
    m = jnp.maximum(jnp.maximum(sks[0], sks[1]), sks[2])
    es = [jnp.exp(sk - m) for sk in sks]
    z = es[0] + es[1] + es[2]
    ws = [e / z for e in es]
    attn = None
    for k in range(_K):
        wf = jnp.concatenate(
            [lax.broadcast_in_dim(ws[k][:, h:h + 1], (_TB, _DH), (0, 1))
             for h in range(_NH)], axis=1)
        contrib = wf * vs[k]
        attn = contrib if attn is None else attn + contrib
    ao = _dott(attn, wo_ref[...]) + bo_ref[...]
    prev = cur if first_step else ctx_ref[...]
    combined = jnp.concatenate([ao, prev], axis=1)
    h1 = _ln(_gelu(_dott(combined, wt1_ref[...]) + bt1_ref[...]),
             ltw_ref[...], ltb_ref[...])
    th = _dott(h1, wt2_ref[...]) + bt2_ref[...]
    oth_ref[...] = th
    octx_ref[...] = th if first_step else 0.7 * th + 0.3 * ctx_ref[...]


def _thought_call(first_step, cur, docs, ctx, wq, bq, wk, bk, wv, bv,
                  wo, bo, wt1, bt1, ltw, ltb, wt2, bt2):
    body = functools.partial(_thought_body, first_step)
    row = pl.BlockSpec((_TB, _D), lambda i: (i, 0))
    drow = pl.BlockSpec((_TB, _K * _D), lambda i: (i, 0))
    wsq = pl.BlockSpec((_D, _D), lambda i: (0, 0))
    wt1s = pl.BlockSpec((_D, 2 * _D), lambda i: (0, 0))
    bsp = pl.BlockSpec((1, _D), lambda i: (0, 0))
    return pl.pallas_call(
        body,
        grid=(_B // _TB,),
        in_specs=[row, drow, row, wsq, bsp, wsq, bsp, wsq, bsp, wsq, bsp,
                  wt1s, bsp, bsp, bsp, wsq, bsp],
        out_specs=(row, row),
        out_shape=(
            jax.ShapeDtypeStruct((_B, _D), jnp.float32),
            jax.ShapeDtypeStruct((_B, _D), jnp.float32),
        ),
    )(cur, docs, ctx, wq, bq.reshape(1, _D), wk, bk.reshape(1, _D),
      wv, bv.reshape(1, _D), wo, bo.reshape(1, _D), wt1,
      bt1.reshape(1, _D), ltw.reshape(1, _D), ltb.reshape(1, _D),
      wt2, bt2.reshape(1, _D))


def _head1_body(x_ref, w_ref, b_ref, o_ref):
    o_ref[...] = _gelu(_dott(x_ref[...], w_ref[...]) + b_ref[...])


def _head1_call(cur, wa1, ba1):
    return pl.pallas_call(
        _head1_body,
        out_shape=jax.ShapeDtypeStruct((_B, _D), jnp.float32),
    )(cur, wa1, ba1.reshape(1, _D))


_VB = 4096
_NVB = (_V + _VB - 1) // _VB


def _head2_body(h_ref, w_ref, b_ref, o_ref):
    o_ref[...] = _dott(h_ref[...], w_ref[...]) + b_ref[...]


def _head2_call(h, wa2, ba2):
    return pl.pallas_call(
        _head2_body,
        grid=(_NVB,),
        in_specs=[
            pl.BlockSpec((_B, _D), lambda c: (0, 0)),
            pl.BlockSpec((_VB, _D), lambda c: (c, 0)),
            pl.BlockSpec((1, _VB), lambda c: (0, c)),
        ],
        out_specs=pl.BlockSpec((_B, _VB), lambda c: (0, c)),
        out_shape=jax.ShapeDtypeStruct((_B, _V), jnp.float32),
    )(h, wa2, ba2.reshape(1, _V))


def kernel(input_repr, W_in1, b_in1, ln_in_w, ln_in_b, Wp1, bp1, Wp2, bp2,
           lnq_w, lnq_b, Wg, bg, attn_in_w, attn_in_b, attn_out_w,
           attn_out_b, Wt1, bt1, lnt_w, lnt_b, Wt2, bt2, Wc1, bc1, Wc2,
           bc2, Wa1, ba1, Wa2, ba2, corpus_keys, corpus_values):
    Wq = attn_in_w[:_D]
    Wk = attn_in_w[_D:2 * _D]
    Wv = attn_in_w[2 * _D:]
    bq = attn_in_b[:_D]
    bk = attn_in_b[_D:2 * _D]
    bv = attn_in_b[2 * _D:]

    current = _input_call(input_repr, W_in1, b_in1, ln_in_w, ln_in_b)
    corpus_norm = _cnorm_call(corpus_keys)
    penalty = jnp.where(jnp.arange(_CPAD) < _C, jnp.float32(0.0),
                        jnp.float32(-1e30)).reshape(1, _CPAD)
    ctx = current
    for step in range(_STEPS):
        q = _qgen_call(step > 0, current, ctx, Wp1, bp1, Wp2, bp2,
                       lnq_w, lnq_b, Wg, bg)
        idx = _simtop_call(q, corpus_norm, penalty)
        docs = _sc_gather(corpus_values, idx.reshape(_NROWS))
        docs = docs.reshape(_B, _K * _D)
        current, ctx = _thought_call(step == 0, current, docs, ctx,
                                     Wq, bq, Wk, bk, Wv, bv,
                                     attn_out_w, attn_out_b, Wt1, bt1,
                                     lnt_w, lnt_b, Wt2, bt2)
    h = _head1_call(current, Wa1, ba1)
    return _head2_call(h, Wa2, ba2)
